# Initial kernel scaffold; baseline (speedup 1.0000x reference)
import jax
import jax.numpy as jnp
from jax.experimental import pallas as pl


def kernel(sources, queries, table):
    raise NotImplementedError("write your pallas kernel here")


# speedup vs baseline: 2.4666x; 2.4666x over previous
import functools

import jax
import jax.numpy as jnp
from jax import lax
from jax.experimental import pallas as pl
from jax.experimental.pallas import tpu as pltpu
from jax.experimental.pallas import tpu_sc as plsc

_VOCAB = 1000000
_EMBED = 64
_MAX_IN_SEQ = 200
_SEQ = 50
_LANES = 16

_NC = 2
_NS = 16
_NW = _NC * _NS

_CB = 16
_ROWS = _CB * _SEQ


def _pos_embedding(seq):
    j = jnp.arange(seq, dtype=jnp.float32)[:, None]
    k = jnp.arange(_EMBED, dtype=jnp.float32)[None, :]
    return 1.0 - j / _MAX_IN_SEQ - (k / _EMBED) * (1.0 - 2.0 * j / _MAX_IN_SEQ)


@functools.lru_cache(maxsize=None)
def _make_sc_pool(n_batch, seq, embed):
    assert n_batch % _NW == 0
    batches_per_worker = n_batch // _NW
    assert batches_per_worker % _CB == 0
    n_chunks = batches_per_worker // _CB
    n_groups = _ROWS // _LANES
    n_kc = embed // _LANES

    mesh = plsc.VectorSubcoreMesh(core_axis_name="c", subcore_axis_name="s")

    @functools.partial(
        pl.kernel,
        mesh=mesh,
        compiler_params=pltpu.CompilerParams(use_tc_tiling_on_sc=False),
        out_type=jax.ShapeDtypeStruct((n_batch, embed), jnp.float32),
        scratch_types=[
            pltpu.VMEM((_ROWS,), jnp.int32),
            pltpu.VMEM((_ROWS, embed), jnp.float32),
            pltpu.VMEM((seq, embed), jnp.float32),
            pltpu.VMEM((_CB, embed), jnp.float32),
            pltpu.SemaphoreType.DMA,
        ],
    )
    def sc_pool(ids_hbm, pos_hbm, table_hbm, out_hbm,
                idx_v, rows_v, pos_v, outbuf_v, sem):
        wid = lax.axis_index("s") * _NC + lax.axis_index("c")
        pltpu.sync_copy(pos_hbm, pos_v)

        def chunk_body(ci, carry):
            batch0 = wid * batches_per_worker + ci * _CB
            row0 = batch0 * seq
            pltpu.sync_copy(ids_hbm.at[pl.ds(row0, _ROWS)], idx_v)
            pltpu.async_copy(table_hbm.at[idx_v], rows_v, sem).wait()

            def flag_body(g, orv):
                idv = idx_v[pl.ds(g * _LANES, _LANES)]
                return orv | jnp.where(idv == 0, 1, 0).astype(jnp.int32)

            orv = lax.fori_loop(0, n_groups, flag_body,
                                jnp.zeros((_LANES,), jnp.int32))
            any_masked = orv[0]
            for lane in range(1, _LANES):
                any_masked = any_masked | orv[lane]

            @pl.when(any_masked > 0)
            def _zero_rows():
                def mask_body(g, carry):
                    idv = idx_v[pl.ds(g * _LANES, _LANES)]
                    for lane in range(_LANES):
                        mf = jnp.where(idv[lane] != 0,
                                       jnp.float32(1.0), jnp.float32(0.0))
                        mv = jnp.full((_LANES,), mf, jnp.float32)
                        r = g * _LANES + lane
                        for c in range(n_kc):
                            colz = pl.ds(c * _LANES, _LANES)
                            rows_v[r, colz] = rows_v[r, colz] * mv
                    return carry

                lax.fori_loop(0, n_groups, mask_body, 0)

            for kc in range(n_kc):
                col = pl.ds(kc * _LANES, _LANES)
                pos_k = [pos_v[j, col] for j in range(seq)]

                def b_body(b, carry, col=col, pos_k=pos_k):
                    r0 = b * seq
                    acc = pos_k[0] * rows_v[r0, col]
                    for j in range(1, seq):
                        acc = acc + pos_k[j] * rows_v[r0 + j, col]
                    outbuf_v[b, col] = acc
                    return carry

                lax.fori_loop(0, _CB, b_body, 0)

            pltpu.sync_copy(outbuf_v, out_hbm.at[pl.ds(batch0, _CB), :])
            return carry

        lax.fori_loop(0, n_chunks, chunk_body, 0)

    return sc_pool


def kernel(sources, queries, table):
    b_src, seq = sources.shape
    b_q = queries.shape[0]
    ids = jnp.concatenate(
        [sources.astype(jnp.int32), queries.astype(jnp.int32)], axis=0
    ).reshape(-1)
    pos = _pos_embedding(seq)
    pooled = _make_sc_pool(b_src + b_q, seq, table.shape[1])(ids, pos, table)
    return pooled[:b_src], pooled[b_src:]


# speedup vs baseline: 2.7145x; 1.1005x over previous
import functools

import jax
import jax.numpy as jnp
from jax import lax
from jax.experimental import pallas as pl
from jax.experimental.pallas import tpu as pltpu
from jax.experimental.pallas import tpu_sc as plsc

_VOCAB = 1000000
_EMBED = 64
_MAX_IN_SEQ = 200
_SEQ = 50
_LANES = 16

_NC = 2
_NS = 16
_NW = _NC * _NS

_CB = 16
_ROWS = _CB * _SEQ


def _pos_embedding(seq):
    j = jnp.arange(seq, dtype=jnp.float32)[:, None]
    k = jnp.arange(_EMBED, dtype=jnp.float32)[None, :]
    return 1.0 - j / _MAX_IN_SEQ - (k / _EMBED) * (1.0 - 2.0 * j / _MAX_IN_SEQ)


@functools.lru_cache(maxsize=None)
def _make_sc_pool(n_batch, seq, embed):
    assert n_batch % _NW == 0
    batches_per_worker = n_batch // _NW
    assert batches_per_worker % _CB == 0
    n_chunks = batches_per_worker // _CB
    assert n_chunks % 2 == 0
    n_groups = _ROWS // _LANES
    n_kc = embed // _LANES

    mesh = plsc.VectorSubcoreMesh(core_axis_name="c", subcore_axis_name="s")

    @functools.partial(
        pl.kernel,
        mesh=mesh,
        compiler_params=pltpu.CompilerParams(use_tc_tiling_on_sc=False),
        out_type=jax.ShapeDtypeStruct((n_batch, embed), jnp.float32),
        scratch_types=[
            pltpu.VMEM((_ROWS,), jnp.int32),
            pltpu.VMEM((_ROWS,), jnp.int32),
            pltpu.VMEM((_ROWS, embed), jnp.float32),
            pltpu.VMEM((_ROWS, embed), jnp.float32),
            pltpu.VMEM((seq, embed), jnp.float32),
            pltpu.VMEM((_CB, embed), jnp.float32),
            pltpu.SemaphoreType.DMA,
            pltpu.SemaphoreType.DMA,
        ],
    )
    def sc_pool(ids_hbm, pos_hbm, table_hbm, out_hbm,
                idx0_v, idx1_v, rows0_v, rows1_v, pos_v, outbuf_v,
                sem0, sem1):
        wid = lax.axis_index("s") * _NC + lax.axis_index("c")
        batch_base = wid * batches_per_worker
        pltpu.sync_copy(pos_hbm, pos_v)

        idx_bufs = (idx0_v, idx1_v)
        rows_bufs = (rows0_v, rows1_v)
        sems = (sem0, sem1)

        def stage(ci, par):
            row0 = (batch_base + ci * _CB) * seq
            pltpu.sync_copy(ids_hbm.at[pl.ds(row0, _ROWS)], idx_bufs[par])
            pltpu.make_async_copy(
                table_hbm.at[idx_bufs[par]], rows_bufs[par], sems[par]
            ).start()

        def process(ci, par):
            idx_v = idx_bufs[par]
            rows_v = rows_bufs[par]

            @pl.when(ci + 1 < n_chunks)
            def _stage_next():
                stage(ci + 1, par ^ 1)

            def flag_body(g, orv):
                idv = idx_v[pl.ds(g * _LANES, _LANES)]
                return orv | jnp.where(idv == 0, 1, 0).astype(jnp.int32)

            orv = lax.fori_loop(0, n_groups, flag_body,
                                jnp.zeros((_LANES,), jnp.int32))
            any_masked = orv[0]
            for lane in range(1, _LANES):
                any_masked = any_masked | orv[lane]

            pltpu.make_async_copy(
                table_hbm.at[idx_v], rows_v, sems[par]
            ).wait()

            @pl.when(any_masked > 0)
            def _zero_rows():
                def mask_body(g, carry):
                    idv = idx_v[pl.ds(g * _LANES, _LANES)]
                    for lane in range(_LANES):
                        mf = jnp.where(idv[lane] != 0,
                                       jnp.float32(1.0), jnp.float32(0.0))
                        mv = jnp.full((_LANES,), mf, jnp.float32)
                        r = g * _LANES + lane
                        for c in range(n_kc):
                            colz = pl.ds(c * _LANES, _LANES)
                            rows_v[r, colz] = rows_v[r, colz] * mv
                    return carry

                lax.fori_loop(0, n_groups, mask_body, 0)

            for kc in range(n_kc):
                col = pl.ds(kc * _LANES, _LANES)
                pos_k = [pos_v[j, col] for j in range(seq)]

                def b_body(b, carry, col=col, pos_k=pos_k, rows_v=rows_v):
                    r0 = b * seq
                    acc = pos_k[0] * rows_v[r0, col]
                    for j in range(1, seq):
                        acc = acc + pos_k[j] * rows_v[r0 + j, col]
                    outbuf_v[b, col] = acc
                    return carry

                lax.fori_loop(0, _CB, b_body, 0)

            pltpu.sync_copy(outbuf_v,
                            out_hbm.at[pl.ds(batch_base + ci * _CB, _CB), :])

        stage(0, 0)

        def pair_body(g, carry):
            process(2 * g, 0)
            process(2 * g + 1, 1)
            return carry

        lax.fori_loop(0, n_chunks // 2, pair_body, 0)

    return sc_pool


def kernel(sources, queries, table):
    b_src, seq = sources.shape
    b_q = queries.shape[0]
    ids = jnp.concatenate(
        [sources.astype(jnp.int32), queries.astype(jnp.int32)], axis=0
    ).reshape(-1)
    pos = _pos_embedding(seq)
    pooled = _make_sc_pool(b_src + b_q, seq, table.shape[1])(ids, pos, table)
    return pooled[:b_src], pooled[b_src:]


# speedup vs baseline: 2.7608x; 1.0171x over previous
import functools

import jax
import jax.numpy as jnp
from jax import lax
from jax.experimental import pallas as pl
from jax.experimental.pallas import tpu as pltpu
from jax.experimental.pallas import tpu_sc as plsc

_VOCAB = 1000000
_EMBED = 64
_MAX_IN_SEQ = 200
_SEQ = 50
_LANES = 16

_NC = 2
_NS = 16
_NW = _NC * _NS

_CB = 8
_ROWS = _CB * _SEQ


def _pos_embedding(seq):
    j = jnp.arange(seq, dtype=jnp.float32)[:, None]
    k = jnp.arange(_EMBED, dtype=jnp.float32)[None, :]
    return 1.0 - j / _MAX_IN_SEQ - (k / _EMBED) * (1.0 - 2.0 * j / _MAX_IN_SEQ)


@functools.lru_cache(maxsize=None)
def _make_sc_pool(n_batch, seq, embed):
    assert n_batch % _NW == 0
    batches_per_worker = n_batch // _NW
    assert batches_per_worker % _CB == 0
    n_chunks = batches_per_worker // _CB
    assert n_chunks % 2 == 0
    n_groups = _ROWS // _LANES
    n_kc = embed // _LANES

    mesh = plsc.VectorSubcoreMesh(core_axis_name="c", subcore_axis_name="s")

    @functools.partial(
        pl.kernel,
        mesh=mesh,
        compiler_params=pltpu.CompilerParams(use_tc_tiling_on_sc=True),
        out_type=jax.ShapeDtypeStruct((n_batch, 2 * embed), jnp.float32),
        scratch_types=[
            pltpu.VMEM((_ROWS,), jnp.int32),
            pltpu.VMEM((_ROWS,), jnp.int32),
            pltpu.VMEM((_ROWS, 2 * embed), jnp.float32),
            pltpu.VMEM((_ROWS, 2 * embed), jnp.float32),
            pltpu.VMEM((seq, embed), jnp.float32),
            pltpu.VMEM((_CB, 2 * embed), jnp.float32),
            pltpu.SemaphoreType.DMA,
            pltpu.SemaphoreType.DMA,
        ],
    )
    def sc_pool(ids_hbm, pos_hbm, table_hbm, out_hbm,
                idx0_v, idx1_v, rows0_v, rows1_v, pos_v, outbuf_v,
                sem0, sem1):
        wid = lax.axis_index("s") * _NC + lax.axis_index("c")
        batch_base = wid * batches_per_worker
        pltpu.sync_copy(pos_hbm, pos_v)

        idx_bufs = (idx0_v, idx1_v)
        rows_bufs = (rows0_v, rows1_v)
        sems = (sem0, sem1)

        def stage(ci, par):
            row0 = (batch_base + ci * _CB) * seq
            pltpu.sync_copy(ids_hbm.at[pl.ds(row0, _ROWS)], idx_bufs[par])
            pltpu.make_async_copy(
                table_hbm.at[idx_bufs[par]], rows_bufs[par], sems[par]
            ).start()

        def process(ci, par):
            idx_v = idx_bufs[par]
            rows_v = rows_bufs[par]

            @pl.when(ci + 1 < n_chunks)
            def _stage_next():
                stage(ci + 1, par ^ 1)

            def flag_body(g, orv):
                idv = idx_v[pl.ds(g * _LANES, _LANES)]
                return orv | jnp.where(idv == 0, 1, 0).astype(jnp.int32)

            orv = lax.fori_loop(0, n_groups, flag_body,
                                jnp.zeros((_LANES,), jnp.int32))
            any_masked = orv[0]
            for lane in range(1, _LANES):
                any_masked = any_masked | orv[lane]

            pltpu.make_async_copy(
                table_hbm.at[idx_v], rows_v, sems[par]
            ).wait()

            @pl.when(any_masked > 0)
            def _zero_rows():
                def mask_body(g, carry):
                    idv = idx_v[pl.ds(g * _LANES, _LANES)]
                    for lane in range(_LANES):
                        mf = jnp.where(idv[lane] != 0,
                                       jnp.float32(1.0), jnp.float32(0.0))
                        mv = jnp.full((_LANES,), mf, jnp.float32)
                        r = g * _LANES + lane
                        for c in range(n_kc):
                            colz = pl.ds(c * _LANES, _LANES)
                            rows_v[r, colz] = rows_v[r, colz] * mv
                    return carry

                lax.fori_loop(0, n_groups, mask_body, 0)

            for kc in range(n_kc):
                col = pl.ds(kc * _LANES, _LANES)
                pos_k = [pos_v[j, col] for j in range(seq)]

                def b_body(b, carry, col=col, pos_k=pos_k, rows_v=rows_v):
                    r0 = b * seq
                    acc = pos_k[0] * rows_v[r0, col]
                    for j in range(1, seq):
                        acc = acc + pos_k[j] * rows_v[r0 + j, col]
                    outbuf_v[b, col] = acc
                    return carry

                lax.fori_loop(0, _CB, b_body, 0)

            pltpu.sync_copy(outbuf_v,
                            out_hbm.at[pl.ds(batch_base + ci * _CB, _CB), :])

        stage(0, 0)

        def pair_body(g, carry):
            process(2 * g, 0)
            process(2 * g + 1, 1)
            return carry

        lax.fori_loop(0, n_chunks // 2, pair_body, 0)

    return sc_pool


def kernel(sources, queries, table):
    b_src, seq = sources.shape
    b_q = queries.shape[0]
    ids = jnp.concatenate(
        [sources.astype(jnp.int32), queries.astype(jnp.int32)], axis=0
    ).reshape(-1)
    pos = _pos_embedding(seq)
    tbl = jnp.pad(table, ((0, 0), (0, table.shape[1])))
    pooled = _make_sc_pool(b_src + b_q, seq, table.shape[1])(ids, pos, tbl)
    return pooled[:b_src, :_EMBED], pooled[b_src:, :_EMBED]


# speedup vs baseline: 3.0025x; 1.0875x over previous
import functools

import jax
import jax.numpy as jnp
from jax import lax
from jax.experimental import pallas as pl
from jax.experimental.pallas import tpu as pltpu
from jax.experimental.pallas import tpu_sc as plsc

_VOCAB = 1000000
_EMBED = 64
_MAX_IN_SEQ = 200
_SEQ = 50
_LANES = 16

_NC = 2
_NS = 16
_NW = _NC * _NS

_CB = 8
_ROWS = _CB * _SEQ


def _pos_embedding(seq):
    j = jnp.arange(seq, dtype=jnp.float32)[:, None]
    k = jnp.arange(_EMBED, dtype=jnp.float32)[None, :]
    return 1.0 - j / _MAX_IN_SEQ - (k / _EMBED) * (1.0 - 2.0 * j / _MAX_IN_SEQ)


@functools.lru_cache(maxsize=None)
def _make_sc_pool(n_batch, seq, embed):
    assert n_batch % _NW == 0
    batches_per_worker = n_batch // _NW
    assert batches_per_worker % _CB == 0
    n_chunks = batches_per_worker // _CB
    assert n_chunks % 2 == 0
    n_groups = _ROWS // _LANES
    n_kc = embed // _LANES

    mesh = plsc.VectorSubcoreMesh(core_axis_name="c", subcore_axis_name="s")

    @functools.partial(
        pl.kernel,
        mesh=mesh,
        compiler_params=pltpu.CompilerParams(use_tc_tiling_on_sc=True),
        out_type=jax.ShapeDtypeStruct((n_batch, 2 * embed), jnp.float32),
        scratch_types=[
            pltpu.VMEM((_ROWS,), jnp.int32),
            pltpu.VMEM((_ROWS,), jnp.int32),
            pltpu.VMEM((_ROWS, 2 * embed), jnp.float32),
            pltpu.VMEM((_ROWS, 2 * embed), jnp.float32),
            pltpu.VMEM((seq, embed), jnp.float32),
            pltpu.VMEM((_CB, 2 * embed), jnp.float32),
            pltpu.VMEM((_CB, 2 * embed), jnp.float32),
            pltpu.SemaphoreType.DMA,
            pltpu.SemaphoreType.DMA,
            pltpu.SemaphoreType.DMA,
            pltpu.SemaphoreType.DMA,
        ],
    )
    def sc_pool(ids_hbm, pos_hbm, table_hbm, out_hbm,
                idx0_v, idx1_v, rows0_v, rows1_v, pos_v, outbuf0_v, outbuf1_v,
                sem0, sem1, osem0, osem1):
        wid = lax.axis_index("s") * _NC + lax.axis_index("c")
        batch_base = wid * batches_per_worker
        pltpu.sync_copy(pos_hbm, pos_v)

        idx_bufs = (idx0_v, idx1_v)
        rows_bufs = (rows0_v, rows1_v)
        sems = (sem0, sem1)
        out_bufs = (outbuf0_v, outbuf1_v)
        osems = (osem0, osem1)

        def stage(ci, par):
            row0 = (batch_base + ci * _CB) * seq
            pltpu.sync_copy(ids_hbm.at[pl.ds(row0, _ROWS)], idx_bufs[par])
            pltpu.make_async_copy(
                table_hbm.at[idx_bufs[par]], rows_bufs[par], sems[par]
            ).start()

        def process(ci, par):
            idx_v = idx_bufs[par]
            rows_v = rows_bufs[par]
            outbuf_v = out_bufs[par]

            @pl.when(ci + 1 < n_chunks)
            def _stage_next():
                stage(ci + 1, par ^ 1)

            def flag_body(g, orv):
                idv = idx_v[pl.ds(g * _LANES, _LANES)]
                return orv | jnp.where(idv == 0, 1, 0).astype(jnp.int32)

            orv = lax.fori_loop(0, n_groups, flag_body,
                                jnp.zeros((_LANES,), jnp.int32))
            any_masked = orv[0]
            for lane in range(1, _LANES):
                any_masked = any_masked | orv[lane]

            pltpu.make_async_copy(
                table_hbm.at[idx_v], rows_v, sems[par]
            ).wait()

            @pl.when(any_masked > 0)
            def _zero_rows():
                def mask_body(g, carry):
                    idv = idx_v[pl.ds(g * _LANES, _LANES)]
                    for lane in range(_LANES):
                        mf = jnp.where(idv[lane] != 0,
                                       jnp.float32(1.0), jnp.float32(0.0))
                        mv = jnp.full((_LANES,), mf, jnp.float32)
                        r = g * _LANES + lane
                        for c in range(n_kc):
                            colz = pl.ds(c * _LANES, _LANES)
                            rows_v[r, colz] = rows_v[r, colz] * mv
                    return carry

                lax.fori_loop(0, n_groups, mask_body, 0)

            @pl.when(ci >= 2)
            def _drain_out():
                pltpu.make_async_copy(
                    outbuf_v,
                    out_hbm.at[pl.ds(batch_base + (ci - 2) * _CB, _CB), :],
                    osems[par],
                ).wait()

            for kc in range(n_kc):
                col = pl.ds(kc * _LANES, _LANES)
                pos_k = [pos_v[j, col] for j in range(seq)]

                def b_body(b, carry, col=col, pos_k=pos_k, rows_v=rows_v,
                           outbuf_v=outbuf_v):
                    r0 = b * seq
                    acc0 = pos_k[0] * rows_v[r0, col]
                    acc1 = pos_k[1] * rows_v[r0 + 1, col]
                    for j in range(2, seq - 1, 2):
                        acc0 = acc0 + pos_k[j] * rows_v[r0 + j, col]
                        acc1 = acc1 + pos_k[j + 1] * rows_v[r0 + j + 1, col]
                    outbuf_v[b, col] = acc0 + acc1
                    return carry

                lax.fori_loop(0, _CB, b_body, 0)

            pltpu.make_async_copy(
                outbuf_v,
                out_hbm.at[pl.ds(batch_base + ci * _CB, _CB), :],
                osems[par],
            ).start()

        stage(0, 0)

        def pair_body(g, carry):
            process(2 * g, 0)
            process(2 * g + 1, 1)
            return carry

        lax.fori_loop(0, n_chunks // 2, pair_body, 0)

        for par in (0, 1):
            pltpu.make_async_copy(
                out_bufs[par],
                out_hbm.at[pl.ds(batch_base + (n_chunks - 2 + par) * _CB,
                                 _CB), :],
                osems[par],
            ).wait()

    return sc_pool


def kernel(sources, queries, table):
    b_src, seq = sources.shape
    b_q = queries.shape[0]
    ids = jnp.concatenate(
        [sources.astype(jnp.int32), queries.astype(jnp.int32)], axis=0
    ).reshape(-1)
    pos = _pos_embedding(seq)
    tbl = jnp.pad(table, ((0, 0), (0, table.shape[1])))
    pooled = _make_sc_pool(b_src + b_q, seq, table.shape[1])(ids, pos, tbl)
    return pooled[:b_src, :_EMBED], pooled[b_src:, :_EMBED]


# speedup vs baseline: 3.0520x; 1.0165x over previous
import functools

import jax
import jax.numpy as jnp
from jax import lax
from jax.experimental import pallas as pl
from jax.experimental.pallas import tpu as pltpu
from jax.experimental.pallas import tpu_sc as plsc

_VOCAB = 1000000
_EMBED = 64
_MAX_IN_SEQ = 200
_SEQ = 50
_LANES = 16

_NC = 2
_NS = 16
_NW = _NC * _NS

_CB = 8
_ROWS = _CB * _SEQ


def _pos_embedding(seq):
    j = jnp.arange(seq, dtype=jnp.float32)[:, None]
    k = jnp.arange(_EMBED, dtype=jnp.float32)[None, :]
    return 1.0 - j / _MAX_IN_SEQ - (k / _EMBED) * (1.0 - 2.0 * j / _MAX_IN_SEQ)


@functools.lru_cache(maxsize=None)
def _make_sc_pool(n_batch, seq, embed):
    assert n_batch % _NW == 0
    batches_per_worker = n_batch // _NW
    assert batches_per_worker % _CB == 0
    n_chunks = batches_per_worker // _CB
    assert n_chunks % 2 == 0
    n_groups = _ROWS // _LANES
    n_kc = embed // _LANES

    mesh = plsc.VectorSubcoreMesh(core_axis_name="c", subcore_axis_name="s")

    @functools.partial(
        pl.kernel,
        mesh=mesh,
        compiler_params=pltpu.CompilerParams(use_tc_tiling_on_sc=True),
        out_type=jax.ShapeDtypeStruct((n_batch, 2 * embed), jnp.float32),
        scratch_types=[
            pltpu.VMEM((_ROWS,), jnp.int32),
            pltpu.VMEM((_ROWS,), jnp.int32),
            pltpu.VMEM((_ROWS,), jnp.int32),
            pltpu.VMEM((_ROWS,), jnp.int32),
            pltpu.VMEM((_ROWS, 2 * embed), jnp.float32),
            pltpu.VMEM((_ROWS, 2 * embed), jnp.float32),
            pltpu.VMEM((seq, embed), jnp.float32),
            pltpu.VMEM((_CB, 2 * embed), jnp.float32),
            pltpu.VMEM((_CB, 2 * embed), jnp.float32),
            pltpu.SemaphoreType.DMA,
            pltpu.SemaphoreType.DMA,
            pltpu.SemaphoreType.DMA,
            pltpu.SemaphoreType.DMA,
            pltpu.SemaphoreType.DMA,
            pltpu.SemaphoreType.DMA,
            pltpu.SemaphoreType.DMA,
            pltpu.SemaphoreType.DMA,
        ],
    )
    def sc_pool(ids_hbm, pos_hbm, table_hbm, out_hbm,
                idx0_v, idx1_v, idx2_v, idx3_v, rows0_v, rows1_v, pos_v,
                outbuf0_v, outbuf1_v,
                sem0, sem1, osem0, osem1,
                isem0, isem1, isem2, isem3):
        wid = lax.axis_index("s") * _NC + lax.axis_index("c")
        batch_base = wid * batches_per_worker
        pltpu.sync_copy(pos_hbm, pos_v)

        idx_bufs = (idx0_v, idx1_v, idx2_v, idx3_v)
        rows_bufs = (rows0_v, rows1_v)
        sems = (sem0, sem1)
        out_bufs = (outbuf0_v, outbuf1_v)
        osems = (osem0, osem1)
        isems = (isem0, isem1, isem2, isem3)

        def ids_copy(ci, ip):
            row0 = (batch_base + ci * _CB) * seq
            return pltpu.make_async_copy(
                ids_hbm.at[pl.ds(row0, _ROWS)], idx_bufs[ip], isems[ip])

        def process(ci, par, ipar):
            idx_v = idx_bufs[ipar]
            rows_v = rows_bufs[par]
            outbuf_v = out_bufs[par]

            @pl.when(ci + 2 < n_chunks)
            def _ids_ahead():
                ids_copy(ci + 2, (ipar + 2) % 4).start()

            @pl.when(ci + 1 < n_chunks)
            def _gather_next():
                ids_copy(ci + 1, (ipar + 1) % 4).wait()
                pltpu.make_async_copy(
                    table_hbm.at[idx_bufs[(ipar + 1) % 4]],
                    rows_bufs[par ^ 1], sems[par ^ 1],
                ).start()

            def flag_body(g, orv):
                idv = idx_v[pl.ds(g * _LANES, _LANES)]
                return orv | jnp.where(idv == 0, 1, 0).astype(jnp.int32)

            orv = lax.fori_loop(0, n_groups, flag_body,
                                jnp.zeros((_LANES,), jnp.int32))
            any_masked = orv[0]
            for lane in range(1, _LANES):
                any_masked = any_masked | orv[lane]

            pltpu.make_async_copy(
                table_hbm.at[idx_v], rows_v, sems[par]
            ).wait()

            @pl.when(any_masked > 0)
            def _zero_rows():
                def mask_body(g, carry):
                    idv = idx_v[pl.ds(g * _LANES, _LANES)]
                    for lane in range(_LANES):
                        mf = jnp.where(idv[lane] != 0,
                                       jnp.float32(1.0), jnp.float32(0.0))
                        mv = jnp.full((_LANES,), mf, jnp.float32)
                        r = g * _LANES + lane
                        for c in range(n_kc):
                            colz = pl.ds(c * _LANES, _LANES)
                            rows_v[r, colz] = rows_v[r, colz] * mv
                    return carry

                lax.fori_loop(0, n_groups, mask_body, 0)

            @pl.when(ci >= 2)
            def _drain_out():
                pltpu.make_async_copy(
                    outbuf_v,
                    out_hbm.at[pl.ds(batch_base + (ci - 2) * _CB, _CB), :],
                    osems[par],
                ).wait()

            for kc in range(n_kc):
                col = pl.ds(kc * _LANES, _LANES)
                pos_k = [pos_v[j, col] for j in range(seq)]

                def b_body(b, carry, col=col, pos_k=pos_k, rows_v=rows_v,
                           outbuf_v=outbuf_v):
                    r0 = b * seq
                    acc0 = pos_k[0] * rows_v[r0, col]
                    acc1 = pos_k[1] * rows_v[r0 + 1, col]
                    for j in range(2, seq - 1, 2):
                        acc0 = acc0 + pos_k[j] * rows_v[r0 + j, col]
                        acc1 = acc1 + pos_k[j + 1] * rows_v[r0 + j + 1, col]
                    outbuf_v[b, col] = acc0 + acc1
                    return carry

                lax.fori_loop(0, _CB, b_body, 0)

            pltpu.make_async_copy(
                outbuf_v,
                out_hbm.at[pl.ds(batch_base + ci * _CB, _CB), :],
                osems[par],
            ).start()

        ids_copy(0, 0).start()
        ids_copy(1, 1).start()
        ids_copy(0, 0).wait()
        pltpu.make_async_copy(
            table_hbm.at[idx_bufs[0]], rows_bufs[0], sems[0]).start()

        def quad_body(g, carry):
            for q in range(4):
                process(4 * g + q, q % 2, q)
            return carry

        lax.fori_loop(0, n_chunks // 4, quad_body, 0)

        for par in (0, 1):
            pltpu.make_async_copy(
                out_bufs[par],
                out_hbm.at[pl.ds(batch_base + (n_chunks - 2 + par) * _CB,
                                 _CB), :],
                osems[par],
            ).wait()

    return sc_pool


def kernel(sources, queries, table):
    b_src, seq = sources.shape
    b_q = queries.shape[0]
    ids = jnp.concatenate(
        [sources.astype(jnp.int32), queries.astype(jnp.int32)], axis=0
    ).reshape(-1)
    pos = _pos_embedding(seq)
    tbl = jnp.pad(table, ((0, 0), (0, table.shape[1])))
    pooled = _make_sc_pool(b_src + b_q, seq, table.shape[1])(ids, pos, tbl)
    return pooled[:b_src, :_EMBED], pooled[b_src:, :_EMBED]
